# baseline (speedup 1.0000x reference)
import jax
import jax.numpy as jnp
from jax import lax
from jax.experimental import pallas as pl
from jax.experimental.pallas import tpu as pltpu
from jax.experimental.pallas import tpu_sc as plsc

NUM_WORKERS = 32
BATCH = 16384
B_PER_W = BATCH // NUM_WORKERS
EMBED = 64
LANES = 16

NUM_ROWS = 1000000
PK_CHUNK = 160
N_PK = NUM_ROWS // PK_CHUNK

N_ROUNDS = 8
ROUND = B_PER_W // N_ROUNDS


def _pack_body(utab_hbm, itab_hbm, upk_hbm, ipk_hbm,
               ua, ub, ia, ib, sem):
    wid = lax.axis_index("s") * 2 + lax.axis_index("c")

    def repack(src_a, dst_b):
        def row_body(r2, _):
            for c in range(8):
                v = src_a[2 * r2 + c // 4, pl.ds((c % 4) * LANES, LANES)]
                dst_b[r2, pl.ds(c * LANES, LANES)] = v
            return 0

        lax.fori_loop(0, PK_CHUNK // 2, row_body, 0)

    def chunk_body(k, _):
        cid = k * NUM_WORKERS + wid

        @pl.when(cid < N_PK)
        def _():
            src = pl.ds(cid * PK_CHUNK, PK_CHUNK)
            dst = pl.ds(cid * (PK_CHUNK // 2), PK_CHUNK // 2)
            cu = pltpu.async_copy(utab_hbm.at[src], ua, sem)
            ci = pltpu.async_copy(itab_hbm.at[src], ia, sem)
            cu.wait()
            ci.wait()
            repack(ua, ub)
            repack(ia, ib)
            cu = pltpu.async_copy(ub, upk_hbm.at[dst], sem)
            ci = pltpu.async_copy(ib, ipk_hbm.at[dst], sem)
            cu.wait()
            ci.wait()

        return 0

    lax.fori_loop(0, pl.cdiv(N_PK, NUM_WORKERS), chunk_body, 0)


def _gather_dot_body(uid_hbm, iid_hbm, upk_hbm, ipk_hbm, out_hbm,
                     gid_u, gid_i, sid_u, sid_i, u_buf, i_buf, out_v, sem):
    wid = lax.axis_index("s") * 2 + lax.axis_index("c")
    iota = lax.iota(jnp.int32, LANES)

    for j in range(N_ROUNDS):
        pltpu.sync_copy(
            uid_hbm.at[pl.ds(wid * B_PER_W + j * ROUND, ROUND)], gid_u.at[j])
        pltpu.sync_copy(
            iid_hbm.at[pl.ds(wid * B_PER_W + j * ROUND, ROUND)], gid_i.at[j])

    def split_body(j, _):
        r = j // (ROUND // LANES)
        c = j % (ROUND // LANES)
        sl = pl.ds(c * LANES, LANES)
        for gid, sid in ((gid_u, sid_u), (gid_i, sid_i)):
            v = gid[r, sl]
            hi = (v >= NUM_ROWS // 2).astype(jnp.int32)
            sid[r, sl] = hi * EMBED
            gid[r, sl] = v - hi * (NUM_ROWS // 2)
        return 0

    lax.fori_loop(0, B_PER_W // LANES, split_body, 0)

    def round_body(r, _):
        cu = pltpu.async_copy(upk_hbm.at[gid_u.at[r]], u_buf, sem)
        ci = pltpu.async_copy(ipk_hbm.at[gid_i.at[r]], i_buf, sem)
        cu.wait()
        ci.wait()

        def chunk_body(c, _):
            sl = pl.ds(c * LANES, LANES)
            evec = c * LANES + iota
            base_u = sid_u[r, sl]
            base_i = sid_i[r, sl]
            acc = jnp.zeros((LANES,), jnp.float32)
            cvec = iota
            for _d in range(EMBED):
                u = plsc.load_gather(u_buf, [evec, base_u + cvec])
                v = plsc.load_gather(i_buf, [evec, base_i + cvec])
                acc = acc + u * v
                cvec = (cvec + 1) & (EMBED - 1)
            out_v[pl.ds(r * ROUND + c * LANES, LANES)] = acc
            return 0

        lax.fori_loop(0, ROUND // LANES, chunk_body, 0)
        return 0

    lax.fori_loop(0, N_ROUNDS, round_body, 0)

    pltpu.sync_copy(out_v, out_hbm.at[pl.ds(wid * B_PER_W, B_PER_W)])


@jax.jit
def kernel(user_ids, item_ids, user_table, item_table):
    uids = user_ids.astype(jnp.int32)
    iids = item_ids.astype(jnp.int32)
    half = NUM_ROWS // 2
    upk = jnp.concatenate([user_table[:half], user_table[half:]], axis=1)
    ipk = jnp.concatenate([item_table[:half], item_table[half:]], axis=1)
    mesh = plsc.VectorSubcoreMesh(core_axis_name="c", subcore_axis_name="s")
    params = pltpu.CompilerParams(needs_layout_passes=False)

    gather_dot = pl.kernel(
        _gather_dot_body,
        out_type=jax.ShapeDtypeStruct((BATCH,), jnp.float32),
        mesh=mesh,
        compiler_params=params,
        scratch_types=[
            pltpu.VMEM((N_ROUNDS, ROUND), jnp.int32),
            pltpu.VMEM((N_ROUNDS, ROUND), jnp.int32),
            pltpu.VMEM((N_ROUNDS, ROUND), jnp.int32),
            pltpu.VMEM((N_ROUNDS, ROUND), jnp.int32),
            pltpu.VMEM((ROUND, 2 * EMBED), jnp.float32),
            pltpu.VMEM((ROUND, 2 * EMBED), jnp.float32),
            pltpu.VMEM((B_PER_W,), jnp.float32),
            pltpu.SemaphoreType.DMA,
        ],
    )
    return gather_dot(uids, iids, upk, ipk)


# speedup vs baseline: 2.1345x; 2.1345x over previous
import jax
import jax.numpy as jnp
from jax import lax
from jax.experimental import pallas as pl
from jax.experimental.pallas import tpu as pltpu
from jax.experimental.pallas import tpu_sc as plsc

NUM_WORKERS = 32
BATCH = 16384
B_PER_W = BATCH // NUM_WORKERS
EMBED = 64
LANES = 16
N_SEMS = 4

N_ROUNDS = 8
ROUND = B_PER_W // N_ROUNDS


def _body(uid_hbm, iid_hbm, utab_hbm, itab_hbm, out_hbm,
          idx_u, idx_i, u_buf, i_buf, out_v, *sems):
    wid = lax.axis_index("s") * 2 + lax.axis_index("c")
    iota = lax.iota(jnp.int32, LANES)

    pltpu.sync_copy(uid_hbm.at[pl.ds(wid * B_PER_W, B_PER_W)], idx_u)
    pltpu.sync_copy(iid_hbm.at[pl.ds(wid * B_PER_W, B_PER_W)], idx_i)

    def round_body(r, _):
        base = r * ROUND

        def fire_body(g, _):
            uvec = idx_u[pl.ds(base + g * LANES, LANES)]
            ivec = idx_i[pl.ds(base + g * LANES, LANES)]
            for l in range(LANES):
                j = g * LANES + l
                sem = sems[l % N_SEMS]
                pltpu.async_copy(utab_hbm.at[uvec[l]], u_buf.at[j], sem)
                pltpu.async_copy(itab_hbm.at[ivec[l]], i_buf.at[j], sem)
            return 0

        lax.fori_loop(0, ROUND // LANES, fire_body, 0)
        n = 2 * (ROUND // N_SEMS)
        for q in range(N_SEMS):
            pltpu.make_async_copy(
                utab_hbm.at[pl.ds(0, n)], u_buf.at[pl.ds(0, n)], sems[q]
            ).wait()

        def chunk_body(c, _):
            evec = c * LANES + iota
            acc = jnp.zeros((LANES,), jnp.float32)
            cvec = iota
            for _d in range(EMBED):
                u = plsc.load_gather(u_buf, [evec, cvec])
                v = plsc.load_gather(i_buf, [evec, cvec])
                acc = acc + u * v
                cvec = (cvec + 1) & (EMBED - 1)
            out_v[pl.ds(base + c * LANES, LANES)] = acc
            return 0

        lax.fori_loop(0, ROUND // LANES, chunk_body, 0)
        return 0

    lax.fori_loop(0, N_ROUNDS, round_body, 0)

    pltpu.sync_copy(out_v, out_hbm.at[pl.ds(wid * B_PER_W, B_PER_W)])


@jax.jit
def kernel(user_ids, item_ids, user_table, item_table):
    uids = user_ids.astype(jnp.int32)
    iids = item_ids.astype(jnp.int32)
    mesh = plsc.VectorSubcoreMesh(core_axis_name="c", subcore_axis_name="s")
    run = pl.kernel(
        _body,
        out_type=jax.ShapeDtypeStruct((BATCH,), jnp.float32),
        mesh=mesh,
        compiler_params=pltpu.CompilerParams(needs_layout_passes=False),
        scratch_types=[
            pltpu.VMEM((B_PER_W,), jnp.int32),
            pltpu.VMEM((B_PER_W,), jnp.int32),
            pltpu.VMEM((ROUND, EMBED), jnp.float32),
            pltpu.VMEM((ROUND, EMBED), jnp.float32),
            pltpu.VMEM((B_PER_W,), jnp.float32),
        ] + [pltpu.SemaphoreType.DMA] * N_SEMS,
    )
    return run(uids, iids, user_table, item_table)
